# baseline (speedup 1.0000x reference)
import functools

import jax
import jax.numpy as jnp
from jax import lax
from jax.experimental import pallas as pl
from jax.experimental.pallas import tpu as pltpu
from jax.experimental.pallas import tpu_sc as plsc

N = 100000
E = 3200000
D_IN = 128
D_HID = 16
N_CLS = 2

NC = 2
NS = 16
NW = NC * NS

CHUNK = 128
CPW = 808
E_PAD = NW * CPW * CHUNK
HGRP = 4
NPAIR = CPW // (2 * HGRP)
DGRP = 8
NDIT = CPW // DGRP

SLAB = 6272
N_PAD = NS * SLAB

_mesh = plsc.VectorSubcoreMesh(
    core_axis_name="c", subcore_axis_name="s", num_cores=NC, num_subcores=NS)


def _deg_body(dstg, out, outx, dst_v, ones_v, zbuf, chunk_v, exp_v, acc,
              idx_sem, sct_sem):
    cid = lax.axis_index("c")
    sid = lax.axis_index("s")
    wid = cid * NS + sid
    base = sid * SLAB

    def issue_idx(it, par):
        pltpu.async_copy(dstg.at[wid, pl.ds(it * DGRP, DGRP)], dst_v.at[par],
                         idx_sem)

    def wait_idx():
        pltpu.make_async_copy(dstg.at[wid, pl.ds(0, DGRP)], dst_v.at[0],
                              idx_sem).wait()

    def drain_sct():
        for c in range(DGRP):
            pltpu.make_async_copy(ones_v, acc.at[dst_v.at[0, c]],
                                  sct_sem).wait()

    issue_idx(0, 0)

    @pl.loop(0, 64)
    def _fill(i):
        zbuf[pl.ds(i * 16, 16)] = jnp.zeros((16,), jnp.float32)

    @pl.loop(0, CHUNK // 16)
    def _fill1(i):
        ones_v[pl.ds(i * 16, 16)] = jnp.ones((16,), jnp.float32)

    for k in range(6):
        pltpu.sync_copy(zbuf, acc.at[pl.ds(base + k * 1024, 1024)])
    pltpu.sync_copy(zbuf.at[pl.ds(0, 128)], acc.at[pl.ds(base + 6144, 128)])
    plsc.subcore_barrier()

    @pl.loop(0, NDIT)
    def _it(u):
        p = lax.rem(u, 2)
        wait_idx()

        @pl.when(u > 0)
        def _():
            drain_sct()

        @pl.when(u + 1 < NDIT)
        def _():
            issue_idx(u + 1, 1 - p)

        for c in range(DGRP):
            pltpu.async_copy(ones_v, acc.at[dst_v.at[p, c]], sct_sem, add=True)

    drain_sct()
    plsc.subcore_barrier()
    pltpu.sync_copy(acc.at[pl.ds(base, SLAB)], out.at[cid, pl.ds(base, SLAB)])

    for k in range(SLAB // 784):
        pltpu.sync_copy(acc.at[pl.ds(base + k * 784, 784)], chunk_v)

        @pl.loop(0, 784)
        def _exp(i):
            exp_v[i] = plsc.load_gather(
                chunk_v, [jnp.full((16,), i, jnp.int32)])

        pltpu.sync_copy(exp_v, outx.at[cid, pl.ds(base + k * 784, 784)])


_deg_call = pl.kernel(
    _deg_body,
    out_type=[
        jax.ShapeDtypeStruct((NC, N_PAD), jnp.float32),
        jax.ShapeDtypeStruct((NC, N_PAD, D_HID), jnp.float32),
    ],
    compiler_params=pltpu.CompilerParams(needs_layout_passes=False),
    mesh=_mesh,
    scratch_types=[
        pltpu.VMEM((2, DGRP, CHUNK), jnp.int32),
        pltpu.VMEM((CHUNK,), jnp.float32),
        pltpu.VMEM((1024,), jnp.float32),
        pltpu.VMEM((784,), jnp.float32),
        pltpu.VMEM((784, D_HID), jnp.float32),
        pltpu.VMEM_SHARED((N_PAD,), jnp.float32),
        pltpu.SemaphoreType.DMA,
        pltpu.SemaphoreType.DMA,
    ],
)


def _agg_body(y_hbm, srcg, dstg, out, src_v, dst_v, rows_v, zrows, acc,
              idx_sem, gat_sem0, gat_sem1, sct_sem):
    cid = lax.axis_index("c")
    sid = lax.axis_index("s")
    wid = cid * NS + sid
    base = sid * SLAB

    def issue_idx(pair, par):
        for h in range(2):
            pltpu.async_copy(srcg.at[wid, pl.ds(pair * 2 * HGRP + h * HGRP,
                                                HGRP)],
                             src_v.at[par, h], idx_sem)
            pltpu.async_copy(dstg.at[wid, pl.ds(pair * 2 * HGRP + h * HGRP,
                                                HGRP)],
                             dst_v.at[par, h], idx_sem)

    def wait_idx():
        for _ in range(4):
            pltpu.make_async_copy(srcg.at[wid, pl.ds(0, HGRP)],
                                  src_v.at[0, 0], idx_sem).wait()

    def drain_sct():
        for h in range(2):
            for c in range(HGRP):
                pltpu.make_async_copy(rows_v.at[h, c],
                                      acc.at[dst_v.at[0, h, c]],
                                      sct_sem).wait()

    issue_idx(0, 0)

    @pl.loop(0, 64)
    def _fill(i):
        zrows[i] = jnp.zeros((16,), jnp.float32)

    for k in range(98):
        pltpu.sync_copy(zrows, acc.at[pl.ds(base + k * 64, 64)])
    plsc.subcore_barrier()

    @pl.loop(0, NPAIR)
    def _pair(u):
        p = lax.rem(u, 2)
        wait_idx()

        @pl.when(u > 0)
        def _():
            drain_sct()

        @pl.when(u + 1 < NPAIR)
        def _():
            issue_idx(u + 1, 1 - p)

        gd0 = [pltpu.async_copy(y_hbm.at[src_v.at[p, 0, c]], rows_v.at[0, c],
                                gat_sem0) for c in range(HGRP)]
        gd1 = [pltpu.async_copy(y_hbm.at[src_v.at[p, 1, c]], rows_v.at[1, c],
                                gat_sem1) for c in range(HGRP)]
        for d in gd0:
            d.wait()
        for c in range(HGRP):
            pltpu.async_copy(rows_v.at[0, c], acc.at[dst_v.at[p, 0, c]],
                             sct_sem, add=True)
        for d in gd1:
            d.wait()
        for c in range(HGRP):
            pltpu.async_copy(rows_v.at[1, c], acc.at[dst_v.at[p, 1, c]],
                             sct_sem, add=True)

    drain_sct()
    plsc.subcore_barrier()
    pltpu.sync_copy(acc.at[pl.ds(base, SLAB)], out.at[cid, pl.ds(base, SLAB)])


_agg_call = pl.kernel(
    _agg_body,
    out_type=jax.ShapeDtypeStruct((NC, N_PAD, D_HID), jnp.float32),
    compiler_params=pltpu.CompilerParams(use_tc_tiling_on_sc=False),
    mesh=_mesh,
    scratch_types=[
        pltpu.VMEM((2, 2, HGRP, CHUNK), jnp.int32),
        pltpu.VMEM((2, 2, HGRP, CHUNK), jnp.int32),
        pltpu.VMEM((2, HGRP, CHUNK, D_HID), jnp.float32),
        pltpu.VMEM((64, D_HID), jnp.float32),
        pltpu.VMEM_SHARED((N_PAD, D_HID), jnp.float32),
        pltpu.SemaphoreType.DMA,
        pltpu.SemaphoreType.DMA,
        pltpu.SemaphoreType.DMA,
        pltpu.SemaphoreType.DMA,
    ],
)


_BR = 10000
_G = N // _BR


def _stage_a_body(x_ref, w_ref, d0_ref, d1_ref, y1_ref):
    deg = d0_ref[...] + d1_ref[...]
    dis = lax.rsqrt(deg)
    xw = jnp.dot(x_ref[...], w_ref[...], preferred_element_type=jnp.float32)
    y1_ref[...] = xw * dis


def _stage_a(x, w1, d0, d1):
    return pl.pallas_call(
        _stage_a_body,
        grid=(_G,),
        in_specs=[
            pl.BlockSpec((_BR, D_IN), lambda i: (i, 0)),
            pl.BlockSpec((D_IN, D_HID), lambda i: (0, 0)),
            pl.BlockSpec((_BR, 1), lambda i: (i, 0)),
            pl.BlockSpec((_BR, 1), lambda i: (i, 0)),
        ],
        out_specs=pl.BlockSpec((_BR, D_HID), lambda i: (i, 0)),
        out_shape=jax.ShapeDtypeStruct((N, D_HID), jnp.float32),
    )(x, w1, d0, d1)


NPK = N_PAD * D_HID // 128
_BPK = 1568
_GP = NPK // _BPK


def _stage_b_body(p0_ref, p1_ref, dx0_ref, dx1_ref, b1_ref, y2_ref):
    disx = lax.rsqrt(dx0_ref[...] + dx1_ref[...])
    h = jnp.maximum(disx * (p0_ref[...] + p1_ref[...]) + b1_ref[...], 0.0)
    y2_ref[...] = disx * h


def _stage_b(p0, p1, dx0, dx1, b1t):
    return pl.pallas_call(
        _stage_b_body,
        grid=(_GP,),
        in_specs=[
            pl.BlockSpec((_BPK, 128), lambda i: (i, 0)),
            pl.BlockSpec((_BPK, 128), lambda i: (i, 0)),
            pl.BlockSpec((_BPK, 128), lambda i: (i, 0)),
            pl.BlockSpec((_BPK, 128), lambda i: (i, 0)),
            pl.BlockSpec((1, 128), lambda i: (0, 0)),
        ],
        out_specs=pl.BlockSpec((_BPK, 128), lambda i: (i, 0)),
        out_shape=jax.ShapeDtypeStruct((NPK, 128), jnp.float32),
    )(p0, p1, dx0, dx1, b1t)


def _stage_c_body(q0_ref, q1_ref, dx0_ref, dx1_ref, w2e_ref, b2_ref, out_ref):
    disx = lax.rsqrt(dx0_ref[...] + dx1_ref[...])
    z = disx * (q0_ref[...] + q1_ref[...])
    out_ref[...] = jnp.dot(z, w2e_ref[...],
                           preferred_element_type=jnp.float32) + b2_ref[...]


def _stage_c(q0, q1, dx0, dx1, w2e, b2t):
    return pl.pallas_call(
        _stage_c_body,
        grid=(_GP,),
        in_specs=[
            pl.BlockSpec((_BPK, 128), lambda i: (i, 0)),
            pl.BlockSpec((_BPK, 128), lambda i: (i, 0)),
            pl.BlockSpec((_BPK, 128), lambda i: (i, 0)),
            pl.BlockSpec((_BPK, 128), lambda i: (i, 0)),
            pl.BlockSpec((128, 8 * N_CLS), lambda i: (0, 0)),
            pl.BlockSpec((1, 8 * N_CLS), lambda i: (0, 0)),
        ],
        out_specs=pl.BlockSpec((_BPK, 8 * N_CLS), lambda i: (i, 0)),
        out_shape=jax.ShapeDtypeStruct((NPK, 8 * N_CLS), jnp.float32),
    )(q0, q1, dx0, dx1, w2e, b2t)


def kernel(x, edge_index, W1, b1, W2, b2):
    src = edge_index[0].astype(jnp.int32)
    dst = edge_index[1].astype(jnp.int32)
    loop = jnp.arange(N, dtype=jnp.int32)
    pad = E_PAD - E - N
    ar = jnp.arange(pad, dtype=jnp.int32)
    srcg = jnp.concatenate([src, loop, ar % N]).reshape(NW, CPW, CHUNK)
    dstg = jnp.concatenate([dst, loop, N + ar % (N_PAD - N)]).reshape(
        NW, CPW, CHUNK)

    degp, degx = _deg_call(dstg)
    d0 = degp[0, :N].reshape(N, 1)
    d1 = degp[1, :N].reshape(N, 1)
    dxv = degx.reshape(NC, NPK, 128)
    y1 = _stage_a(x, W1, d0, d1)
    b1t = jnp.tile(b1, 128 // D_HID).reshape(1, 128)
    p = _agg_call(y1, srcg, dstg)
    pv = p.reshape(NC, NPK, 128)
    y2v = _stage_b(pv[0], pv[1], dxv[0], dxv[1], b1t)
    q = _agg_call(y2v.reshape(N_PAD, D_HID)[:N], srcg, dstg)
    qv = q.reshape(NC, NPK, 128)
    w2e = jnp.kron(jnp.eye(8, dtype=jnp.float32), W2)
    b2t = jnp.tile(b2, 8).reshape(1, 8 * N_CLS)
    outp = _stage_c(qv[0], qv[1], dxv[0], dxv[1], w2e, b2t)
    return outp.reshape(N_PAD, N_CLS)[:N]


# speedup vs baseline: 1.0348x; 1.0348x over previous
import functools

import jax
import jax.numpy as jnp
from jax import lax
from jax.experimental import pallas as pl
from jax.experimental.pallas import tpu as pltpu
from jax.experimental.pallas import tpu_sc as plsc

N = 100000
E = 3200000
D_IN = 128
D_HID = 16
N_CLS = 2

NC = 2
NS = 16
NW = NC * NS

CHUNK = 128
CPW = 800
E_PAD = NW * CPW * CHUNK
HGRP = 4
NPAIR = CPW // (2 * HGRP)
DGRP = 16
NDIT = CPW // DGRP

SLAB = 6272
N_PAD = NS * SLAB

_mesh = plsc.VectorSubcoreMesh(
    core_axis_name="c", subcore_axis_name="s", num_cores=NC, num_subcores=NS)


def _deg_body(dstg, out, dst_v, ones_v, zbuf, acc, idx_sem, sct_sem):
    cid = lax.axis_index("c")
    sid = lax.axis_index("s")
    wid = cid * NS + sid
    base = sid * SLAB

    def issue_idx(it, par):
        pltpu.async_copy(dstg.at[wid, pl.ds(it * DGRP, DGRP)], dst_v.at[par],
                         idx_sem)

    def wait_idx():
        pltpu.make_async_copy(dstg.at[wid, pl.ds(0, DGRP)], dst_v.at[0],
                              idx_sem).wait()

    def drain_sct():
        for c in range(DGRP):
            pltpu.make_async_copy(ones_v, acc.at[dst_v.at[0, c]],
                                  sct_sem).wait()

    issue_idx(0, 0)

    @pl.loop(0, 64)
    def _fill(i):
        zbuf[pl.ds(i * 16, 16)] = jnp.zeros((16,), jnp.float32)

    @pl.loop(0, CHUNK // 16)
    def _fill1(i):
        ones_v[pl.ds(i * 16, 16)] = jnp.ones((16,), jnp.float32)

    for k in range(6):
        pltpu.sync_copy(zbuf, acc.at[pl.ds(base + k * 1024, 1024)])
    pltpu.sync_copy(zbuf.at[pl.ds(0, 128)], acc.at[pl.ds(base + 6144, 128)])
    plsc.subcore_barrier()

    @pl.loop(0, NDIT)
    def _it(u):
        p = lax.rem(u, 2)
        wait_idx()

        @pl.when(u > 0)
        def _():
            drain_sct()

        @pl.when(u + 1 < NDIT)
        def _():
            issue_idx(u + 1, 1 - p)

        for c in range(DGRP):
            pltpu.async_copy(ones_v, acc.at[dst_v.at[p, c]], sct_sem, add=True)

    drain_sct()
    plsc.subcore_barrier()
    pltpu.sync_copy(acc.at[pl.ds(base, SLAB)], out.at[cid, pl.ds(base, SLAB)])


_deg_call = pl.kernel(
    _deg_body,
    out_type=jax.ShapeDtypeStruct((NC, N_PAD), jnp.float32),
    mesh=_mesh,
    scratch_types=[
        pltpu.VMEM((2, DGRP, CHUNK), jnp.int32),
        pltpu.VMEM((CHUNK,), jnp.float32),
        pltpu.VMEM((1024,), jnp.float32),
        pltpu.VMEM_SHARED((N_PAD,), jnp.float32),
        pltpu.SemaphoreType.DMA,
        pltpu.SemaphoreType.DMA,
    ],
)


def _agg_body(y_hbm, srcg, dstg, out, src_v, dst_v, rows_v, zrows, acc,
              idx_sem, gat_sem0, gat_sem1, sct_sem):
    cid = lax.axis_index("c")
    sid = lax.axis_index("s")
    wid = cid * NS + sid
    base = sid * SLAB

    def issue_idx(pair, par):
        for h in range(2):
            pltpu.async_copy(srcg.at[wid, pl.ds(pair * 2 * HGRP + h * HGRP,
                                                HGRP)],
                             src_v.at[par, h], idx_sem)
            pltpu.async_copy(dstg.at[wid, pl.ds(pair * 2 * HGRP + h * HGRP,
                                                HGRP)],
                             dst_v.at[par, h], idx_sem)

    def wait_idx():
        for _ in range(4):
            pltpu.make_async_copy(srcg.at[wid, pl.ds(0, HGRP)],
                                  src_v.at[0, 0], idx_sem).wait()

    def drain_sct():
        for h in range(2):
            for c in range(HGRP):
                pltpu.make_async_copy(rows_v.at[h, c],
                                      acc.at[dst_v.at[0, h, c]],
                                      sct_sem).wait()

    issue_idx(0, 0)

    @pl.loop(0, 64)
    def _fill(i):
        zrows[i] = jnp.zeros((16,), jnp.float32)

    for k in range(98):
        pltpu.sync_copy(zrows, acc.at[pl.ds(base + k * 64, 64)])
    plsc.subcore_barrier()

    @pl.loop(0, NPAIR)
    def _pair(u):
        p = lax.rem(u, 2)
        wait_idx()

        @pl.when(u > 0)
        def _():
            drain_sct()

        @pl.when(u + 1 < NPAIR)
        def _():
            issue_idx(u + 1, 1 - p)

        gd0 = [pltpu.async_copy(y_hbm.at[src_v.at[p, 0, c]], rows_v.at[0, c],
                                gat_sem0) for c in range(HGRP)]
        gd1 = [pltpu.async_copy(y_hbm.at[src_v.at[p, 1, c]], rows_v.at[1, c],
                                gat_sem1) for c in range(HGRP)]
        for d in gd0:
            d.wait()
        for c in range(HGRP):
            pltpu.async_copy(rows_v.at[0, c], acc.at[dst_v.at[p, 0, c]],
                             sct_sem, add=True)
        for d in gd1:
            d.wait()
        for c in range(HGRP):
            pltpu.async_copy(rows_v.at[1, c], acc.at[dst_v.at[p, 1, c]],
                             sct_sem, add=True)

    drain_sct()
    plsc.subcore_barrier()

    @pl.when(sid < NS - 1)
    def _():
        pltpu.sync_copy(acc.at[pl.ds(base, SLAB)],
                        out.at[cid, pl.ds(base, SLAB)])

    @pl.when(sid == NS - 1)
    def _():
        pltpu.sync_copy(acc.at[pl.ds(base, N - (NS - 1) * SLAB)],
                        out.at[cid, pl.ds(base, N - (NS - 1) * SLAB)])


_agg_call = pl.kernel(
    _agg_body,
    out_type=jax.ShapeDtypeStruct((NC, N, D_HID), jnp.float32),
    compiler_params=pltpu.CompilerParams(use_tc_tiling_on_sc=False),
    mesh=_mesh,
    scratch_types=[
        pltpu.VMEM((2, 2, HGRP, CHUNK), jnp.int32),
        pltpu.VMEM((2, 2, HGRP, CHUNK), jnp.int32),
        pltpu.VMEM((2, HGRP, CHUNK, D_HID), jnp.float32),
        pltpu.VMEM((64, D_HID), jnp.float32),
        pltpu.VMEM_SHARED((N_PAD, D_HID), jnp.float32),
        pltpu.SemaphoreType.DMA,
        pltpu.SemaphoreType.DMA,
        pltpu.SemaphoreType.DMA,
        pltpu.SemaphoreType.DMA,
    ],
)


_BR = 10000
_G = N // _BR


def _stage_a_body(x_ref, w_ref, d0_ref, d1_ref, y1_ref, dis_ref):
    deg = d0_ref[...] + d1_ref[...] + 1.0
    dis = lax.rsqrt(deg)
    xw = jnp.dot(x_ref[...], w_ref[...], preferred_element_type=jnp.float32)
    y1_ref[...] = xw * dis
    dis_ref[...] = dis


def _stage_a(x, w1, d0, d1):
    return pl.pallas_call(
        _stage_a_body,
        grid=(_G,),
        in_specs=[
            pl.BlockSpec((_BR, D_IN), lambda i: (i, 0)),
            pl.BlockSpec((D_IN, D_HID), lambda i: (0, 0)),
            pl.BlockSpec((_BR, 1), lambda i: (i, 0)),
            pl.BlockSpec((_BR, 1), lambda i: (i, 0)),
        ],
        out_specs=[
            pl.BlockSpec((_BR, D_HID), lambda i: (i, 0)),
            pl.BlockSpec((_BR, 1), lambda i: (i, 0)),
        ],
        out_shape=[
            jax.ShapeDtypeStruct((N, D_HID), jnp.float32),
            jax.ShapeDtypeStruct((N, 1), jnp.float32),
        ],
    )(x, w1, d0, d1)


EF = N * D_HID
_BF = 204800
_GF = (EF + _BF - 1) // _BF


def _stage_b_body(p0_ref, p1_ref, y1_ref, dx_ref, b1_ref, y2_ref):
    d = dx_ref[...]
    h = jnp.maximum(d * (p0_ref[...] + p1_ref[...] + y1_ref[...])
                    + b1_ref[...], 0.0)
    y2_ref[...] = d * h


def _stage_b(p0f, p1f, y1f, disxf, b1f):
    return pl.pallas_call(
        _stage_b_body,
        grid=(_GF,),
        in_specs=[
            pl.BlockSpec((_BF,), lambda i: (i,)),
            pl.BlockSpec((_BF,), lambda i: (i,)),
            pl.BlockSpec((_BF,), lambda i: (i,)),
            pl.BlockSpec((_BF,), lambda i: (i,)),
            pl.BlockSpec((_BF,), lambda i: (0,)),
        ],
        out_specs=pl.BlockSpec((_BF,), lambda i: (i,)),
        out_shape=jax.ShapeDtypeStruct((EF,), jnp.float32),
    )(p0f, p1f, y1f, disxf, b1f)


def _stage_c1_body(q0_ref, q1_ref, y2_ref, dx_ref, z_ref):
    z_ref[...] = dx_ref[...] * (q0_ref[...] + q1_ref[...] + y2_ref[...])


def _stage_c1(q0f, q1f, y2f, disxf):
    return pl.pallas_call(
        _stage_c1_body,
        grid=(_GF,),
        in_specs=[
            pl.BlockSpec((_BF,), lambda i: (i,)),
            pl.BlockSpec((_BF,), lambda i: (i,)),
            pl.BlockSpec((_BF,), lambda i: (i,)),
            pl.BlockSpec((_BF,), lambda i: (i,)),
        ],
        out_specs=pl.BlockSpec((_BF,), lambda i: (i,)),
        out_shape=jax.ShapeDtypeStruct((EF,), jnp.float32),
    )(q0f, q1f, y2f, disxf)


def _stage_c2_body(z_ref, w2_ref, b2_ref, out_ref):
    out_ref[...] = jnp.dot(z_ref[...], w2_ref[...],
                           preferred_element_type=jnp.float32) + b2_ref[...]


def _stage_c2(z, w2, b2):
    return pl.pallas_call(
        _stage_c2_body,
        grid=(_G,),
        in_specs=[
            pl.BlockSpec((_BR, D_HID), lambda i: (i, 0)),
            pl.BlockSpec((D_HID, N_CLS), lambda i: (0, 0)),
            pl.BlockSpec((1, N_CLS), lambda i: (0, 0)),
        ],
        out_specs=pl.BlockSpec((_BR, N_CLS), lambda i: (i, 0)),
        out_shape=jax.ShapeDtypeStruct((N, N_CLS), jnp.float32),
    )(z, w2, b2)


def kernel(x, edge_index, W1, b1, W2, b2):
    src = edge_index[0].astype(jnp.int32)
    dst = edge_index[1].astype(jnp.int32)
    pad = E_PAD - E
    ar = jnp.arange(pad, dtype=jnp.int32)
    srcg = jnp.concatenate([src, ar % N]).reshape(NW, CPW, CHUNK)
    dstg = jnp.concatenate([dst, N + ar % (N_PAD - N)]).reshape(NW, CPW, CHUNK)

    degp = _deg_call(dstg)
    d0 = degp[0, :N].reshape(N, 1)
    d1 = degp[1, :N].reshape(N, 1)
    y1, dis = _stage_a(x, W1, d0, d1)
    y1f = y1.reshape(EF)
    disxf = jnp.broadcast_to(dis, (N, D_HID)).reshape(EF)
    b1f = jnp.tile(b1, _BF // D_HID)
    p = _agg_call(y1, srcg, dstg)
    y2f = _stage_b(p[0].reshape(EF), p[1].reshape(EF), y1f, disxf, b1f)
    q = _agg_call(y2f.reshape(N, D_HID), srcg, dstg)
    zf = _stage_c1(q[0].reshape(EF), q[1].reshape(EF), y2f, disxf)
    return _stage_c2(zf.reshape(N, D_HID), W2, b2.reshape(1, N_CLS))


# speedup vs baseline: 1.1696x; 1.1303x over previous
import functools

import jax
import jax.numpy as jnp
from jax import lax
from jax.experimental import pallas as pl
from jax.experimental.pallas import tpu as pltpu
from jax.experimental.pallas import tpu_sc as plsc

N = 100000
E = 3200000
D_IN = 128
D_HID = 16
N_CLS = 2

NC = 2
NS = 16
NW = NC * NS

CHUNK = 128
CPW = 800
E_PAD = NW * CPW * CHUNK
HGRP = 4
NPAIR = CPW // (2 * HGRP)
DGRP = 16
NDIT = CPW // DGRP

SLAB = 6272
N_PAD = NS * SLAB

_mesh = plsc.VectorSubcoreMesh(
    core_axis_name="c", subcore_axis_name="s", num_cores=NC, num_subcores=NS)


def _deg_body(dstg, out, dst_v, ones_v, zbuf, acc, idx_sem, sct_sem):
    cid = lax.axis_index("c")
    sid = lax.axis_index("s")
    wid = cid * NS + sid
    base = sid * SLAB

    def issue_idx(it, par):
        pltpu.async_copy(dstg.at[wid, pl.ds(it * DGRP, DGRP)], dst_v.at[par],
                         idx_sem)

    def wait_idx():
        pltpu.make_async_copy(dstg.at[wid, pl.ds(0, DGRP)], dst_v.at[0],
                              idx_sem).wait()

    def drain_sct():
        for c in range(DGRP):
            pltpu.make_async_copy(ones_v, acc.at[dst_v.at[0, c]],
                                  sct_sem).wait()

    issue_idx(0, 0)

    @pl.loop(0, 64)
    def _fill(i):
        zbuf[pl.ds(i * 16, 16)] = jnp.zeros((16,), jnp.float32)

    @pl.loop(0, CHUNK // 16)
    def _fill1(i):
        ones_v[pl.ds(i * 16, 16)] = jnp.ones((16,), jnp.float32)

    for k in range(6):
        pltpu.sync_copy(zbuf, acc.at[pl.ds(base + k * 1024, 1024)])
    pltpu.sync_copy(zbuf.at[pl.ds(0, 128)], acc.at[pl.ds(base + 6144, 128)])
    plsc.subcore_barrier()

    @pl.loop(0, NDIT)
    def _it(u):
        p = lax.rem(u, 2)
        wait_idx()

        @pl.when(u > 0)
        def _():
            drain_sct()

        @pl.when(u + 1 < NDIT)
        def _():
            issue_idx(u + 1, 1 - p)

        for c in range(DGRP):
            pltpu.async_copy(ones_v, acc.at[dst_v.at[p, c]], sct_sem, add=True)

    drain_sct()
    plsc.subcore_barrier()
    pltpu.sync_copy(acc.at[pl.ds(base, SLAB)], out.at[cid, pl.ds(base, SLAB)])


_deg_call = pl.kernel(
    _deg_body,
    out_type=jax.ShapeDtypeStruct((NC, N_PAD), jnp.float32),
    mesh=_mesh,
    scratch_types=[
        pltpu.VMEM((2, DGRP, CHUNK), jnp.int32),
        pltpu.VMEM((CHUNK,), jnp.float32),
        pltpu.VMEM((1024,), jnp.float32),
        pltpu.VMEM_SHARED((N_PAD,), jnp.float32),
        pltpu.SemaphoreType.DMA,
        pltpu.SemaphoreType.DMA,
    ],
)


def _agg_body(y_hbm, srcg, dstg, out, src_v, dst_v, rows_v, zrows, acc,
              idx_sem, gat_sem0, gat_sem1, sct_sem):
    cid = lax.axis_index("c")
    sid = lax.axis_index("s")
    wid = cid * NS + sid
    base = sid * SLAB

    def issue_idx(pair, par):
        for h in range(2):
            pltpu.async_copy(srcg.at[wid, pl.ds(pair * 2 * HGRP + h * HGRP,
                                                HGRP)],
                             src_v.at[par, h], idx_sem)
            pltpu.async_copy(dstg.at[wid, pl.ds(pair * 2 * HGRP + h * HGRP,
                                                HGRP)],
                             dst_v.at[par, h], idx_sem)

    def wait_idx():
        for _ in range(4):
            pltpu.make_async_copy(srcg.at[wid, pl.ds(0, HGRP)],
                                  src_v.at[0, 0], idx_sem).wait()

    def drain_sct():
        for h in range(2):
            for c in range(HGRP):
                pltpu.make_async_copy(rows_v.at[h, c],
                                      acc.at[dst_v.at[0, h, c]],
                                      sct_sem).wait()

    issue_idx(0, 0)

    @pl.loop(0, 64)
    def _fill(i):
        zrows[i] = jnp.zeros((16,), jnp.float32)

    for k in range(98):
        pltpu.sync_copy(zrows, acc.at[pl.ds(base + k * 64, 64)])
    plsc.subcore_barrier()

    @pl.loop(0, NPAIR)
    def _pair(u):
        p = lax.rem(u, 2)
        wait_idx()

        @pl.when(u > 0)
        def _():
            drain_sct()

        @pl.when(u + 1 < NPAIR)
        def _():
            issue_idx(u + 1, 1 - p)

        gd0 = [pltpu.async_copy(y_hbm.at[src_v.at[p, 0, c]], rows_v.at[0, c],
                                gat_sem0) for c in range(HGRP)]
        gd1 = [pltpu.async_copy(y_hbm.at[src_v.at[p, 1, c]], rows_v.at[1, c],
                                gat_sem1) for c in range(HGRP)]
        for d in gd0:
            d.wait()
        for c in range(HGRP):
            pltpu.async_copy(rows_v.at[0, c], acc.at[dst_v.at[p, 0, c]],
                             sct_sem, add=True)
        for d in gd1:
            d.wait()
        for c in range(HGRP):
            pltpu.async_copy(rows_v.at[1, c], acc.at[dst_v.at[p, 1, c]],
                             sct_sem, add=True)

    drain_sct()
    plsc.subcore_barrier()

    @pl.when(sid < NS - 1)
    def _():
        pltpu.sync_copy(acc.at[pl.ds(base, SLAB)],
                        out.at[cid, pl.ds(base, SLAB)])

    @pl.when(sid == NS - 1)
    def _():
        pltpu.sync_copy(acc.at[pl.ds(base, N - (NS - 1) * SLAB)],
                        out.at[cid, pl.ds(base, N - (NS - 1) * SLAB)])


_agg_call = pl.kernel(
    _agg_body,
    out_type=jax.ShapeDtypeStruct((NC, N, D_HID), jnp.float32),
    compiler_params=pltpu.CompilerParams(use_tc_tiling_on_sc=False),
    mesh=_mesh,
    scratch_types=[
        pltpu.VMEM((2, 2, HGRP, CHUNK), jnp.int32),
        pltpu.VMEM((2, 2, HGRP, CHUNK), jnp.int32),
        pltpu.VMEM((2, HGRP, CHUNK, D_HID), jnp.float32),
        pltpu.VMEM((64, D_HID), jnp.float32),
        pltpu.VMEM_SHARED((N_PAD, D_HID), jnp.float32),
        pltpu.SemaphoreType.DMA,
        pltpu.SemaphoreType.DMA,
        pltpu.SemaphoreType.DMA,
        pltpu.SemaphoreType.DMA,
    ],
)


_BR = 10000
_G = N // _BR


def _stage_a_body(x_ref, w_ref, d0_ref, d1_ref, y1_ref, dis_ref):
    deg = d0_ref[...] + d1_ref[...] + 1.0
    dis = lax.rsqrt(deg)
    xw = jnp.dot(x_ref[...], w_ref[...], preferred_element_type=jnp.float32)
    y1_ref[...] = xw * dis
    dis_ref[...] = dis


def _stage_a(x, w1, d0, d1):
    return pl.pallas_call(
        _stage_a_body,
        grid=(_G,),
        in_specs=[
            pl.BlockSpec((_BR, D_IN), lambda i: (i, 0)),
            pl.BlockSpec((D_IN, D_HID), lambda i: (0, 0)),
            pl.BlockSpec((_BR, 1), lambda i: (i, 0)),
            pl.BlockSpec((_BR, 1), lambda i: (i, 0)),
        ],
        out_specs=[
            pl.BlockSpec((_BR, D_HID), lambda i: (i, 0)),
            pl.BlockSpec((_BR, 1), lambda i: (i, 0)),
        ],
        out_shape=[
            jax.ShapeDtypeStruct((N, D_HID), jnp.float32),
            jax.ShapeDtypeStruct((N, 1), jnp.float32),
        ],
    )(x, w1, d0, d1)


NPK = N * D_HID // 128
_BPK = 1600
_GP = (NPK + _BPK - 1) // _BPK


def _lane_expand(disb):
    i0 = lax.broadcasted_iota(jnp.int32, (8, 128), 0)
    i1 = lax.broadcasted_iota(jnp.int32, (8, 128), 1)
    r = (i1 // D_HID == i0).astype(jnp.float32)
    return jnp.dot(disb, r, preferred_element_type=jnp.float32)


def _stage_b_body(p0_ref, p1_ref, y1_ref, disb_ref, b1_ref, y2_ref):
    disx = _lane_expand(disb_ref[...])
    h = jnp.maximum(disx * (p0_ref[...] + p1_ref[...] + y1_ref[...])
                    + b1_ref[...], 0.0)
    y2_ref[...] = disx * h


def _stage_b(p0, p1, y1v, disb, b1t):
    return pl.pallas_call(
        _stage_b_body,
        grid=(_GP,),
        in_specs=[
            pl.BlockSpec((_BPK, 128), lambda i: (i, 0)),
            pl.BlockSpec((_BPK, 128), lambda i: (i, 0)),
            pl.BlockSpec((_BPK, 128), lambda i: (i, 0)),
            pl.BlockSpec((_BPK, 8), lambda i: (i, 0)),
            pl.BlockSpec((1, 128), lambda i: (0, 0)),
        ],
        out_specs=pl.BlockSpec((_BPK, 128), lambda i: (i, 0)),
        out_shape=jax.ShapeDtypeStruct((NPK, 128), jnp.float32),
    )(p0, p1, y1v, disb, b1t)


def _stage_c_body(q0_ref, q1_ref, y2_ref, disb_ref, w2e_ref, b2_ref, out_ref):
    disx = _lane_expand(disb_ref[...])
    z = disx * (q0_ref[...] + q1_ref[...] + y2_ref[...])
    out_ref[...] = jnp.dot(z, w2e_ref[...],
                           preferred_element_type=jnp.float32) + b2_ref[...]


def _stage_c(q0, q1, y2v, disb, w2e, b2t):
    return pl.pallas_call(
        _stage_c_body,
        grid=(_GP,),
        in_specs=[
            pl.BlockSpec((_BPK, 128), lambda i: (i, 0)),
            pl.BlockSpec((_BPK, 128), lambda i: (i, 0)),
            pl.BlockSpec((_BPK, 128), lambda i: (i, 0)),
            pl.BlockSpec((_BPK, 8), lambda i: (i, 0)),
            pl.BlockSpec((128, 8 * N_CLS), lambda i: (0, 0)),
            pl.BlockSpec((1, 8 * N_CLS), lambda i: (0, 0)),
        ],
        out_specs=pl.BlockSpec((_BPK, 8 * N_CLS), lambda i: (i, 0)),
        out_shape=jax.ShapeDtypeStruct((NPK, 8 * N_CLS), jnp.float32),
    )(q0, q1, y2v, disb, w2e, b2t)


def kernel(x, edge_index, W1, b1, W2, b2):
    src = edge_index[0].astype(jnp.int32)
    dst = edge_index[1].astype(jnp.int32)
    pad = E_PAD - E
    ar = jnp.arange(pad, dtype=jnp.int32)
    srcg = jnp.concatenate([src, ar % N]).reshape(NW, CPW, CHUNK)
    dstg = jnp.concatenate([dst, N + ar % (N_PAD - N)]).reshape(NW, CPW, CHUNK)

    degp = _deg_call(dstg)
    d0 = degp[0, :N].reshape(N, 1)
    d1 = degp[1, :N].reshape(N, 1)
    y1, dis = _stage_a(x, W1, d0, d1)
    y1v = y1.reshape(NPK, 128)
    disb = dis.reshape(NPK, 8)
    b1t = jnp.tile(b1, 128 // D_HID).reshape(1, 128)
    p = _agg_call(y1, srcg, dstg)
    pv = p.reshape(NC, NPK, 128)
    y2v = _stage_b(pv[0], pv[1], y1v, disb, b1t)
    q = _agg_call(y2v.reshape(N, D_HID), srcg, dstg)
    qv = q.reshape(NC, NPK, 128)
    w2e = jnp.kron(jnp.eye(8, dtype=jnp.float32), W2)
    b2t = jnp.tile(b2, 8).reshape(1, 8 * N_CLS)
    outp = _stage_c(qv[0], qv[1], y2v, disb, w2e, b2t)
    return outp.reshape(N, N_CLS)
